# Initial kernel scaffold; baseline (speedup 1.0000x reference)
import jax
import jax.numpy as jnp
from jax.experimental import pallas as pl


def kernel(feats, neighbor_idx, W_conv, b_conv, W_lin, b_lin, ln_g, ln_b):
    raise NotImplementedError("write your pallas kernel here")


# speedup vs baseline: 1.5109x; 1.5109x over previous
import jax
import jax.numpy as jnp
from jax import lax
from jax.experimental import pallas as pl
from jax.experimental.pallas import tpu as pltpu
from jax.experimental.pallas import tpu_sc as plsc

_N = 50000
_C = 32
_K = 27

_NC = 2
_NS = 16
_NW = _NC * _NS
_CHUNK = 128
_NCH = 13
_CH = _CHUNK * _NCH
_NPAD = _CH * _NW

_BLKN = 2000
_NBLK = _N // _BLKN
_LNB = 2048
_NLNB = _NPAD // _LNB


def _taps_body(feats_ref, wconv_ref, wlin_ref, out_ref):
    wk = wconv_ref[0]
    wl = wlin_ref[...]
    wp = lax.dot_general(wk, wl, (((1,), (1,)), ((), ())),
                         preferred_element_type=jnp.float32)
    out_ref[0] = jnp.dot(feats_ref[...], wp, preferred_element_type=jnp.float32)


def _sc_body(tbl_hbm, idx_hbm, out_hbm, idx_v, acc_v, sem):
    c = lax.axis_index("c")
    s = lax.axis_index("s")
    wid = s * _NC + c
    base = wid * _CH

    def tap(k, add):
        pltpu.sync_copy(idx_hbm.at[k, wid], idx_v)
        cps = [
            pltpu.async_copy(
                tbl_hbm.at[idx_v.at[j]],
                acc_v.at[pl.ds(j * _CHUNK, _CHUNK)],
                sem,
                add=add,
            )
            for j in range(_NCH)
        ]
        for cp in cps:
            cp.wait()

    tap(0, False)

    def body(k, carry):
        tap(k, True)
        return carry

    lax.fori_loop(1, _K, body, 0)
    pltpu.sync_copy(acc_v, out_hbm.at[pl.ds(base, _CH)])


def _ln_body(h_ref, wlin_ref, bconv_ref, blin_ref, g_ref, b_ref, out_ref):
    bias = lax.dot_general(bconv_ref[...], wlin_ref[...],
                           (((1,), (1,)), ((), ())),
                           preferred_element_type=jnp.float32) + blin_ref[...]
    x = h_ref[...] + bias
    mu = jnp.mean(x, axis=-1, keepdims=True)
    xc = x - mu
    var = jnp.mean(xc * xc, axis=-1, keepdims=True)
    out_ref[...] = xc * lax.rsqrt(var + 1e-5) * g_ref[...] + b_ref[...]


def kernel(feats, neighbor_idx, W_conv, b_conv, W_lin, b_lin, ln_g, ln_b):
    idx = neighbor_idx.astype(jnp.int32)
    offs = (jnp.arange(_K, dtype=jnp.int32) * _N)[:, None]
    idx3 = jnp.pad(idx + offs, ((0, 0), (0, _NPAD - _N))).reshape(
        _K, _NW, _NCH, _CHUNK)

    tbl = pl.pallas_call(
        _taps_body,
        grid=(_NBLK, _K),
        in_specs=[
            pl.BlockSpec((_BLKN, _C), lambda nb, k: (nb, 0)),
            pl.BlockSpec((1, _C, _C), lambda nb, k: (k, 0, 0)),
            pl.BlockSpec((_C, _C), lambda nb, k: (0, 0)),
        ],
        out_specs=pl.BlockSpec((1, _BLKN, _C), lambda nb, k: (k, nb, 0)),
        out_shape=jax.ShapeDtypeStruct((_K, _N, _C), jnp.float32),
    )(feats, W_conv, W_lin)

    h = pl.kernel(
        _sc_body,
        out_type=jax.ShapeDtypeStruct((_NPAD, _C), jnp.float32),
        mesh=plsc.VectorSubcoreMesh(core_axis_name="c", subcore_axis_name="s"),
        compiler_params=pltpu.CompilerParams(use_tc_tiling_on_sc=False),
        scratch_types=[
            pltpu.VMEM((_NCH, _CHUNK), jnp.int32),
            pltpu.VMEM((_CH, _C), jnp.float32),
            pltpu.SemaphoreType.DMA,
        ],
    )(tbl.reshape(_K * _N, _C), idx3)

    out = pl.pallas_call(
        _ln_body,
        grid=(_NLNB,),
        in_specs=[
            pl.BlockSpec((_LNB, _C), lambda i: (i, 0)),
            pl.BlockSpec((_C, _C), lambda i: (0, 0)),
            pl.BlockSpec((1, _C), lambda i: (0, 0)),
            pl.BlockSpec((1, _C), lambda i: (0, 0)),
            pl.BlockSpec((1, _C), lambda i: (0, 0)),
            pl.BlockSpec((1, _C), lambda i: (0, 0)),
        ],
        out_specs=pl.BlockSpec((_LNB, _C), lambda i: (i, 0)),
        out_shape=jax.ShapeDtypeStruct((_NPAD, _C), jnp.float32),
    )(h, W_lin, b_conv.reshape(1, _C), b_lin.reshape(1, _C),
      ln_g.reshape(1, _C), ln_b.reshape(1, _C))

    return out[:_N]


# speedup vs baseline: 2.2934x; 1.5179x over previous
import jax
import jax.numpy as jnp
from jax import lax
from jax.experimental import pallas as pl
from jax.experimental.pallas import tpu as pltpu
from jax.experimental.pallas import tpu_sc as plsc

_N = 50000
_C = 32
_K = 27

_NC = 2
_NS = 16
_NW = _NC * _NS
_CH = 1664
_NPAD = _CH * _NW

_BLKN = 1000
_NBLK = _N // _BLKN
_LNB = 2048
_NLNB = _NPAD // _LNB


def _mat_body(feats_ref, wconv_ref, wlin_ref, out_ref, wcat_ref):
    @pl.when(pl.program_id(0) == 0)
    def _():
        w = lax.dot_general(wconv_ref[...], wlin_ref[...],
                            (((2,), (1,)), ((), ())),
                            preferred_element_type=jnp.float32)
        for k in range(_K):
            wcat_ref[pl.ds(0, _C), pl.ds(k * _C, _C)] = w[k]

    out_ref[...] = jnp.dot(feats_ref[...], wcat_ref[...],
                           preferred_element_type=jnp.float32)


def _sc_body(tbl_hbm, idx_hbm, zero_hbm, out_hbm, idx_v, acc_v, sem):
    c = lax.axis_index("c")
    s = lax.axis_index("s")
    wid = s * _NC + c
    base = wid * _CH

    pltpu.sync_copy(zero_hbm, acc_v)
    pltpu.sync_copy(idx_hbm.at[wid], idx_v)
    cps = [
        pltpu.async_copy(tbl_hbm.at[idx_v.at[k]], acc_v, sem, add=True)
        for k in range(_K)
    ]
    for cp in cps:
        cp.wait()
    pltpu.sync_copy(acc_v, out_hbm.at[pl.ds(base, _CH)])


def _ln_body(h_ref, wlin_ref, bconv_ref, blin_ref, g_ref, b_ref, out_ref):
    bias = lax.dot_general(bconv_ref[...], wlin_ref[...],
                           (((1,), (1,)), ((), ())),
                           preferred_element_type=jnp.float32) + blin_ref[...]
    x = h_ref[...] + bias
    mu = jnp.mean(x, axis=-1, keepdims=True)
    xc = x - mu
    var = jnp.mean(xc * xc, axis=-1, keepdims=True)
    out_ref[...] = xc * lax.rsqrt(var + 1e-5) * g_ref[...] + b_ref[...]


def kernel(feats, neighbor_idx, W_conv, b_conv, W_lin, b_lin, ln_g, ln_b):
    idx = neighbor_idx.astype(jnp.int32)
    idx2 = idx * _K + jnp.arange(_K, dtype=jnp.int32)[:, None]
    idx3 = jnp.transpose(
        jnp.pad(idx2, ((0, 0), (0, _NPAD - _N))).reshape(_K, _NW, _CH),
        (1, 0, 2))

    tbl = pl.pallas_call(
        _mat_body,
        grid=(_NBLK,),
        in_specs=[
            pl.BlockSpec((_BLKN, _C), lambda i: (i, 0)),
            pl.BlockSpec((_K, _C, _C), lambda i: (0, 0, 0)),
            pl.BlockSpec((_C, _C), lambda i: (0, 0)),
        ],
        out_specs=pl.BlockSpec((_BLKN, _K * _C), lambda i: (i, 0)),
        out_shape=jax.ShapeDtypeStruct((_N, _K * _C), jnp.float32),
        scratch_shapes=[pltpu.VMEM((_C, _K * _C), jnp.float32)],
    )(feats, W_conv, W_lin)

    h = pl.kernel(
        _sc_body,
        out_type=jax.ShapeDtypeStruct((_NPAD, _C), jnp.float32),
        mesh=plsc.VectorSubcoreMesh(core_axis_name="c", subcore_axis_name="s"),
        compiler_params=pltpu.CompilerParams(use_tc_tiling_on_sc=False),
        scratch_types=[
            pltpu.VMEM((_K, _CH), jnp.int32),
            pltpu.VMEM((_CH, _C), jnp.float32),
            pltpu.SemaphoreType.DMA,
        ],
    )(tbl.reshape(_N * _K, _C), idx3, jnp.zeros((_CH, _C), jnp.float32))

    out = pl.pallas_call(
        _ln_body,
        grid=(_NLNB,),
        in_specs=[
            pl.BlockSpec((_LNB, _C), lambda i: (i, 0)),
            pl.BlockSpec((_C, _C), lambda i: (0, 0)),
            pl.BlockSpec((1, _C), lambda i: (0, 0)),
            pl.BlockSpec((1, _C), lambda i: (0, 0)),
            pl.BlockSpec((1, _C), lambda i: (0, 0)),
            pl.BlockSpec((1, _C), lambda i: (0, 0)),
        ],
        out_specs=pl.BlockSpec((_LNB, _C), lambda i: (i, 0)),
        out_shape=jax.ShapeDtypeStruct((_NPAD, _C), jnp.float32),
    )(h, W_lin, b_conv.reshape(1, _C), b_lin.reshape(1, _C),
      ln_g.reshape(1, _C), ln_b.reshape(1, _C))

    return out[:_N]


# speedup vs baseline: 2.8726x; 1.2525x over previous
import jax
import jax.numpy as jnp
from jax import lax
from jax.experimental import pallas as pl
from jax.experimental.pallas import tpu as pltpu
from jax.experimental.pallas import tpu_sc as plsc

_N = 50000
_C = 32
_K = 27

_NC = 2
_NS = 16
_NW = _NC * _NS
_CH = 1664
_NPAD = _CH * _NW

_BLKN = 1000
_NBLK = _N // _BLKN
_LNB = 2048
_NLNB = _NPAD // _LNB


def _mat_body(feats_ref, wconv_ref, wlin_ref, out_ref, wcat_ref):
    @pl.when(pl.program_id(0) == 0)
    def _():
        w = lax.dot_general(wconv_ref[...], wlin_ref[...],
                            (((2,), (1,)), ((), ())),
                            preferred_element_type=jnp.float32)
        for k in range(_K):
            wcat_ref[pl.ds(0, _C), pl.ds(k * _C, _C)] = w[k]

    out_ref[...] = jnp.dot(feats_ref[...], wcat_ref[...],
                           preferred_element_type=jnp.float32).astype(jnp.bfloat16)


def _sc_body(tbl_hbm, idx_hbm, zero_hbm, out_hbm, idx_v, acc_v, sem):
    c = lax.axis_index("c")
    s = lax.axis_index("s")
    wid = s * _NC + c
    base = wid * _CH

    pltpu.sync_copy(zero_hbm, acc_v)
    pltpu.sync_copy(idx_hbm.at[wid], idx_v)
    cps = [
        pltpu.async_copy(tbl_hbm.at[idx_v.at[k]], acc_v, sem, add=False)
        for k in range(_K)
    ]
    for cp in cps:
        cp.wait()
    pltpu.sync_copy(acc_v, out_hbm.at[pl.ds(base, _CH)])


def _ln_body(h_ref, wlin_ref, bconv_ref, blin_ref, g_ref, b_ref, out_ref):
    bias = lax.dot_general(bconv_ref[...], wlin_ref[...],
                           (((1,), (1,)), ((), ())),
                           preferred_element_type=jnp.float32) + blin_ref[...]
    x = h_ref[...].astype(jnp.float32) + bias
    mu = jnp.mean(x, axis=-1, keepdims=True)
    xc = x - mu
    var = jnp.mean(xc * xc, axis=-1, keepdims=True)
    out_ref[...] = xc * lax.rsqrt(var + 1e-5) * g_ref[...] + b_ref[...]


def kernel(feats, neighbor_idx, W_conv, b_conv, W_lin, b_lin, ln_g, ln_b):
    idx = neighbor_idx.astype(jnp.int32)
    idx2 = idx * _K + jnp.arange(_K, dtype=jnp.int32)[:, None]
    idx3 = jnp.transpose(
        jnp.pad(idx2, ((0, 0), (0, _NPAD - _N))).reshape(_K, _NW, _CH),
        (1, 0, 2))

    tbl = pl.pallas_call(
        _mat_body,
        grid=(_NBLK,),
        in_specs=[
            pl.BlockSpec((_BLKN, _C), lambda i: (i, 0)),
            pl.BlockSpec((_K, _C, _C), lambda i: (0, 0, 0)),
            pl.BlockSpec((_C, _C), lambda i: (0, 0)),
        ],
        out_specs=pl.BlockSpec((_BLKN, _K * _C), lambda i: (i, 0)),
        out_shape=jax.ShapeDtypeStruct((_N, _K * _C), jnp.bfloat16),
        scratch_shapes=[pltpu.VMEM((_C, _K * _C), jnp.float32)],
    )(feats, W_conv, W_lin)

    h = pl.kernel(
        _sc_body,
        out_type=jax.ShapeDtypeStruct((_NPAD, _C), jnp.bfloat16),
        mesh=plsc.VectorSubcoreMesh(core_axis_name="c", subcore_axis_name="s"),
        compiler_params=pltpu.CompilerParams(use_tc_tiling_on_sc=False),
        scratch_types=[
            pltpu.VMEM((_K, _CH), jnp.int32),
            pltpu.VMEM((_CH, _C), jnp.bfloat16),
            pltpu.SemaphoreType.DMA,
        ],
    )(tbl.reshape(_N * _K, _C), idx3, jnp.zeros((_CH, _C), jnp.bfloat16))

    out = pl.pallas_call(
        _ln_body,
        grid=(_NLNB,),
        in_specs=[
            pl.BlockSpec((_LNB, _C), lambda i: (i, 0)),
            pl.BlockSpec((_C, _C), lambda i: (0, 0)),
            pl.BlockSpec((1, _C), lambda i: (0, 0)),
            pl.BlockSpec((1, _C), lambda i: (0, 0)),
            pl.BlockSpec((1, _C), lambda i: (0, 0)),
            pl.BlockSpec((1, _C), lambda i: (0, 0)),
        ],
        out_specs=pl.BlockSpec((_LNB, _C), lambda i: (i, 0)),
        out_shape=jax.ShapeDtypeStruct((_NPAD, _C), jnp.float32),
    )(h, W_lin, b_conv.reshape(1, _C), b_lin.reshape(1, _C),
      ln_g.reshape(1, _C), ln_b.reshape(1, _C))

    return out[:_N]
